# baseline (speedup 1.0000x reference)
import jax
import jax.numpy as jnp
from jax import lax
from jax.experimental import pallas as pl
from jax.experimental.pallas import tpu as pltpu
from jax.experimental.pallas import tpu_sc as plsc

M = 1_000_000
DIM = 64
LANES = 16
NW = 32
D_SC = 16
D_TC = DIM - D_SC
TILE = 512
TILES = M // TILE
TAIL = M - TILES * TILE
PAIRS = 31
TC_BLK = 16384
TC_GRID = (M + TC_BLK - 1) // TC_BLK


def _acc_tile(w_ref, buf, obuf, width):
    nk = width // LANES
    wds = [w_ref[D_TC + di] for di in range(D_SC)]
    acc = [jnp.zeros((LANES,), jnp.float32) for _ in range(nk)]
    for di in range(D_SC):
        for k in range(nk):
            acc[k] = acc[k] + buf[di, pl.ds(k * LANES, LANES)] * wds[di]
    for k in range(nk):
        obuf[pl.ds(k * LANES, LANES)] = acc[k]


def _sc_body(x_ref, w_hbm, out_ref,
             buf0, buf1, ob0, ob1, tbuf, tobuf, w_ref,
             si0, si1, so0, so1):
    wid = lax.axis_index("s") * 2 + lax.axis_index("c")
    pltpu.sync_copy(w_hbm, w_ref)

    bufs = (buf0, buf1)
    obufs = (ob0, ob1)
    sin = (si0, si1)
    sout = (so0, so1)

    def tile_base(i):
        return jnp.minimum(wid + i * NW, TILES - 1) * TILE

    def start_in(i, b):
        pltpu.make_async_copy(
            x_ref.at[pl.ds(D_TC, D_SC), pl.ds(tile_base(i), TILE)],
            bufs[b], sin[b],
        ).start()

    def wait_in(b):
        pltpu.make_async_copy(
            x_ref.at[pl.ds(D_TC, D_SC), pl.ds(0, TILE)], bufs[b], sin[b]
        ).wait()

    def start_out(i, b):
        pltpu.make_async_copy(
            obufs[b], out_ref.at[pl.ds(tile_base(i), TILE)], sout[b]
        ).start()

    def wait_out(b):
        pltpu.make_async_copy(
            obufs[b], out_ref.at[pl.ds(0, TILE)], sout[b]
        ).wait()

    start_in(0, 0)

    def step(j, _):
        i0 = j * 2
        i1 = i0 + 1

        wait_in(0)
        start_in(i1, 1)

        @pl.when(j > 0)
        def _():
            wait_out(0)

        _acc_tile(w_ref, buf0, ob0, TILE)
        start_out(i0, 0)

        wait_in(1)

        @pl.when(i1 + 1 < 2 * PAIRS)
        def _():
            start_in(i1 + 1, 0)

        @pl.when(j > 0)
        def _():
            wait_out(1)

        _acc_tile(w_ref, buf1, ob1, TILE)
        start_out(i1, 1)
        return 0

    lax.fori_loop(0, PAIRS, step, 0)
    wait_out(0)
    wait_out(1)

    @pl.when(wid == 2)
    def _():
        base = TILES * TILE
        pltpu.sync_copy(x_ref.at[pl.ds(D_TC, D_SC), pl.ds(base, TAIL)], tbuf)
        _acc_tile(w_ref, tbuf, tobuf, TAIL)
        pltpu.sync_copy(tobuf, out_ref.at[pl.ds(base, TAIL)])


def _sc_partial(xt, w_bcast):
    mesh = plsc.VectorSubcoreMesh(core_axis_name="c", subcore_axis_name="s")
    return pl.kernel(
        _sc_body,
        mesh=mesh,
        out_type=jax.ShapeDtypeStruct((M,), jnp.float32),
        scratch_types=[
            pltpu.VMEM((D_SC, TILE), jnp.float32),
            pltpu.VMEM((D_SC, TILE), jnp.float32),
            pltpu.VMEM((TILE,), jnp.float32),
            pltpu.VMEM((TILE,), jnp.float32),
            pltpu.VMEM((D_SC, TAIL), jnp.float32),
            pltpu.VMEM((TAIL,), jnp.float32),
            pltpu.VMEM((DIM, LANES), jnp.float32),
            pltpu.SemaphoreType.DMA,
            pltpu.SemaphoreType.DMA,
            pltpu.SemaphoreType.DMA,
            pltpu.SemaphoreType.DMA,
        ],
    )(xt, w_bcast)


def _tc_body(w_ref, x_ref, o_ref):
    o_ref[...] = jnp.sum(x_ref[...] * w_ref[...], axis=0)


def _tc_partial(xt, w_col):
    return pl.pallas_call(
        _tc_body,
        grid=(TC_GRID,),
        in_specs=[
            pl.BlockSpec((D_TC, 1), lambda i: (0, 0)),
            pl.BlockSpec((D_TC, TC_BLK), lambda i: (0, i)),
        ],
        out_specs=pl.BlockSpec((TC_BLK,), lambda i: (i,)),
        out_shape=jax.ShapeDtypeStruct((M,), jnp.float32),
    )(w_col, xt)


def kernel(items_emb, user_w):
    m, dim = items_emb.shape
    xt = items_emb.T
    w_bcast = jnp.tile(user_w.reshape(dim, 1), (1, LANES))
    sc_part = _sc_partial(xt, w_bcast)
    tc_part = _tc_partial(xt, user_w.reshape(dim, 1)[:D_TC])
    return sc_part + tc_part


# speedup vs baseline: 1.1364x; 1.1364x over previous
import jax
import jax.numpy as jnp
from jax import lax
from jax.experimental import pallas as pl
from jax.experimental.pallas import tpu as pltpu
from jax.experimental.pallas import tpu_sc as plsc

M = 1_000_000
DIM = 64
LANES = 16
NW = 32
D_SC = 8
D_TC = DIM - D_SC
TILE = 512
TILES = M // TILE
TAIL = M - TILES * TILE
PAIRS = 31
TC_BLK = 16384
TC_GRID = (M + TC_BLK - 1) // TC_BLK


def _acc_tile(w_ref, buf, obuf, width):
    nk = width // LANES
    wds = [w_ref[D_TC + di] for di in range(D_SC)]
    acc = [jnp.zeros((LANES,), jnp.float32) for _ in range(nk)]
    for di in range(D_SC):
        for k in range(nk):
            acc[k] = acc[k] + buf[di, pl.ds(k * LANES, LANES)] * wds[di]
    for k in range(nk):
        obuf[pl.ds(k * LANES, LANES)] = acc[k]


def _sc_body(x_ref, w_hbm, out_ref,
             buf0, buf1, ob0, ob1, tbuf, tobuf, w_ref,
             si0, si1, so0, so1):
    wid = lax.axis_index("s") * 2 + lax.axis_index("c")
    pltpu.sync_copy(w_hbm, w_ref)

    bufs = (buf0, buf1)
    obufs = (ob0, ob1)
    sin = (si0, si1)
    sout = (so0, so1)

    def tile_base(i):
        return jnp.minimum(wid + i * NW, TILES - 1) * TILE

    def start_in(i, b):
        pltpu.make_async_copy(
            x_ref.at[pl.ds(D_TC, D_SC), pl.ds(tile_base(i), TILE)],
            bufs[b], sin[b],
        ).start()

    def wait_in(b):
        pltpu.make_async_copy(
            x_ref.at[pl.ds(D_TC, D_SC), pl.ds(0, TILE)], bufs[b], sin[b]
        ).wait()

    def start_out(i, b):
        pltpu.make_async_copy(
            obufs[b], out_ref.at[pl.ds(tile_base(i), TILE)], sout[b]
        ).start()

    def wait_out(b):
        pltpu.make_async_copy(
            obufs[b], out_ref.at[pl.ds(0, TILE)], sout[b]
        ).wait()

    start_in(0, 0)

    def step(j, _):
        i0 = j * 2
        i1 = i0 + 1

        wait_in(0)
        start_in(i1, 1)

        @pl.when(j > 0)
        def _():
            wait_out(0)

        _acc_tile(w_ref, buf0, ob0, TILE)
        start_out(i0, 0)

        wait_in(1)

        @pl.when(i1 + 1 < 2 * PAIRS)
        def _():
            start_in(i1 + 1, 0)

        @pl.when(j > 0)
        def _():
            wait_out(1)

        _acc_tile(w_ref, buf1, ob1, TILE)
        start_out(i1, 1)
        return 0

    lax.fori_loop(0, PAIRS, step, 0)
    wait_out(0)
    wait_out(1)

    @pl.when(wid == 2)
    def _():
        base = TILES * TILE
        pltpu.sync_copy(x_ref.at[pl.ds(D_TC, D_SC), pl.ds(base, TAIL)], tbuf)
        _acc_tile(w_ref, tbuf, tobuf, TAIL)
        pltpu.sync_copy(tobuf, out_ref.at[pl.ds(base, TAIL)])


def _sc_partial(xt, w_bcast):
    mesh = plsc.VectorSubcoreMesh(core_axis_name="c", subcore_axis_name="s")
    return pl.kernel(
        _sc_body,
        mesh=mesh,
        out_type=jax.ShapeDtypeStruct((M,), jnp.float32),
        scratch_types=[
            pltpu.VMEM((D_SC, TILE), jnp.float32),
            pltpu.VMEM((D_SC, TILE), jnp.float32),
            pltpu.VMEM((TILE,), jnp.float32),
            pltpu.VMEM((TILE,), jnp.float32),
            pltpu.VMEM((D_SC, TAIL), jnp.float32),
            pltpu.VMEM((TAIL,), jnp.float32),
            pltpu.VMEM((DIM, LANES), jnp.float32),
            pltpu.SemaphoreType.DMA,
            pltpu.SemaphoreType.DMA,
            pltpu.SemaphoreType.DMA,
            pltpu.SemaphoreType.DMA,
        ],
    )(xt, w_bcast)


def _tc_body(w_ref, x_ref, o_ref):
    o_ref[...] = jnp.sum(x_ref[...] * w_ref[...], axis=0)


def _tc_partial(xt, w_col):
    return pl.pallas_call(
        _tc_body,
        grid=(TC_GRID,),
        in_specs=[
            pl.BlockSpec((D_TC, 1), lambda i: (0, 0)),
            pl.BlockSpec((D_TC, TC_BLK), lambda i: (0, i)),
        ],
        out_specs=pl.BlockSpec((TC_BLK,), lambda i: (i,)),
        out_shape=jax.ShapeDtypeStruct((M,), jnp.float32),
    )(w_col, xt)


def kernel(items_emb, user_w):
    m, dim = items_emb.shape
    xt = items_emb.T
    w_bcast = jnp.tile(user_w.reshape(dim, 1), (1, LANES))
    sc_part = _sc_partial(xt, w_bcast)
    tc_part = _tc_partial(xt, user_w.reshape(dim, 1)[:D_TC])
    return sc_part + tc_part


# speedup vs baseline: 1.1438x; 1.0064x over previous
import jax
import jax.numpy as jnp
from jax import lax
from jax.experimental import pallas as pl
from jax.experimental.pallas import tpu as pltpu
from jax.experimental.pallas import tpu_sc as plsc

M = 1_000_000
DIM = 64
LANES = 16
NW = 32
D_SC = 8
D_TC = DIM - D_SC
TILE = 1024
CHUNK = 512
TILES = M // TILE
TAIL = M - TILES * TILE
PAIRS = 16
TC_BLK = 16384
TC_GRID = (M + TC_BLK - 1) // TC_BLK


def _acc_tile(w_ref, buf, obuf, width):
    wds = [w_ref[D_TC + di] for di in range(D_SC)]
    for c0 in range(0, width, CHUNK):
        nk = min(CHUNK, width - c0) // LANES
        acc = [jnp.zeros((LANES,), jnp.float32) for _ in range(nk)]
        for di in range(D_SC):
            for k in range(nk):
                acc[k] = acc[k] + (
                    buf[di, pl.ds(c0 + k * LANES, LANES)] * wds[di]
                )
        for k in range(nk):
            obuf[pl.ds(c0 + k * LANES, LANES)] = acc[k]


def _sc_body(x_ref, w_hbm, out_ref,
             buf0, buf1, ob0, ob1, tbuf, tobuf, w_ref,
             si0, si1, so0, so1):
    wid = lax.axis_index("s") * 2 + lax.axis_index("c")
    pltpu.sync_copy(w_hbm, w_ref)

    bufs = (buf0, buf1)
    obufs = (ob0, ob1)
    sin = (si0, si1)
    sout = (so0, so1)

    def tile_base(i):
        return jnp.minimum(wid + i * NW, TILES - 1) * TILE

    def start_in(i, b):
        pltpu.make_async_copy(
            x_ref.at[pl.ds(D_TC, D_SC), pl.ds(tile_base(i), TILE)],
            bufs[b], sin[b],
        ).start()

    def wait_in(b):
        pltpu.make_async_copy(
            x_ref.at[pl.ds(D_TC, D_SC), pl.ds(0, TILE)], bufs[b], sin[b]
        ).wait()

    def start_out(i, b):
        pltpu.make_async_copy(
            obufs[b], out_ref.at[pl.ds(tile_base(i), TILE)], sout[b]
        ).start()

    def wait_out(b):
        pltpu.make_async_copy(
            obufs[b], out_ref.at[pl.ds(0, TILE)], sout[b]
        ).wait()

    start_in(0, 0)

    def step(j, _):
        i0 = j * 2
        i1 = i0 + 1

        wait_in(0)
        start_in(i1, 1)

        @pl.when(j > 0)
        def _():
            wait_out(0)

        _acc_tile(w_ref, buf0, ob0, TILE)
        start_out(i0, 0)

        wait_in(1)

        @pl.when(i1 + 1 < 2 * PAIRS)
        def _():
            start_in(i1 + 1, 0)

        @pl.when(j > 0)
        def _():
            wait_out(1)

        _acc_tile(w_ref, buf1, ob1, TILE)
        start_out(i1, 1)
        return 0

    lax.fori_loop(0, PAIRS, step, 0)
    wait_out(0)
    wait_out(1)

    @pl.when(wid == 2)
    def _():
        base = TILES * TILE
        pltpu.sync_copy(x_ref.at[pl.ds(D_TC, D_SC), pl.ds(base, TAIL)], tbuf)
        _acc_tile(w_ref, tbuf, tobuf, TAIL)
        pltpu.sync_copy(tobuf, out_ref.at[pl.ds(base, TAIL)])


def _sc_partial(xt, w_bcast):
    mesh = plsc.VectorSubcoreMesh(core_axis_name="c", subcore_axis_name="s")
    return pl.kernel(
        _sc_body,
        mesh=mesh,
        out_type=jax.ShapeDtypeStruct((M,), jnp.float32),
        scratch_types=[
            pltpu.VMEM((D_SC, TILE), jnp.float32),
            pltpu.VMEM((D_SC, TILE), jnp.float32),
            pltpu.VMEM((TILE,), jnp.float32),
            pltpu.VMEM((TILE,), jnp.float32),
            pltpu.VMEM((D_SC, TAIL), jnp.float32),
            pltpu.VMEM((TAIL,), jnp.float32),
            pltpu.VMEM((DIM, LANES), jnp.float32),
            pltpu.SemaphoreType.DMA,
            pltpu.SemaphoreType.DMA,
            pltpu.SemaphoreType.DMA,
            pltpu.SemaphoreType.DMA,
        ],
    )(xt, w_bcast)


def _tc_body(w_ref, x_ref, o_ref):
    o_ref[...] = jnp.sum(x_ref[...] * w_ref[...], axis=0)


def _tc_partial(xt, w_col):
    return pl.pallas_call(
        _tc_body,
        grid=(TC_GRID,),
        in_specs=[
            pl.BlockSpec((D_TC, 1), lambda i: (0, 0)),
            pl.BlockSpec((D_TC, TC_BLK), lambda i: (0, i)),
        ],
        out_specs=pl.BlockSpec((TC_BLK,), lambda i: (i,)),
        out_shape=jax.ShapeDtypeStruct((M,), jnp.float32),
    )(w_col, xt)


def kernel(items_emb, user_w):
    m, dim = items_emb.shape
    xt = items_emb.T
    w_bcast = jnp.tile(user_w.reshape(dim, 1), (1, LANES))
    sc_part = _sc_partial(xt, w_bcast)
    tc_part = _tc_partial(xt, user_w.reshape(dim, 1)[:D_TC])
    return sc_part + tc_part


# speedup vs baseline: 1.6759x; 1.4653x over previous
import jax
import jax.numpy as jnp
from jax.experimental import pallas as pl

BLK = 32768


def _tc_body(w_ref, x_ref, o_ref):
    o_ref[...] = jnp.sum(x_ref[...] * w_ref[...], axis=0)


def kernel(items_emb, user_w):
    m, dim = items_emb.shape
    xt = items_emb.T
    w_col = user_w.reshape(dim, 1)
    grid = (m + BLK - 1) // BLK
    out = pl.pallas_call(
        _tc_body,
        grid=(grid,),
        in_specs=[
            pl.BlockSpec((dim, 1), lambda i: (0, 0)),
            pl.BlockSpec((dim, BLK), lambda i: (0, i)),
        ],
        out_specs=pl.BlockSpec((BLK,), lambda i: (i,)),
        out_shape=jax.ShapeDtypeStruct((m,), jnp.float32),
    )(w_col, xt)
    return out
